# baseline (speedup 1.0000x reference)
import functools

import jax
import jax.numpy as jnp
from jax.experimental import pallas as pl

P, V, C = 50000, 5, 128
BP = 2000
F32 = jnp.float32


def _body(x_ref, wd1_ref, wn1_ref, b1_ref, wd2_ref, wn2_ref, b2_ref,
          wf1_ref, bf1_ref, g_ref, beta_ref, wf2_ref, bf2_ref, out_ref):
    xb = x_ref[...]
    hv = [xb[:, v * C:(v + 1) * C] for v in range(V)]

    wd1 = wd1_ref[...]
    agg1 = jnp.dot(sum(hv), wn1_ref[...], preferred_element_type=F32) + b1_ref[...]
    h1 = [jnp.maximum(jnp.dot(h, wd1, preferred_element_type=F32) + agg1, 0.0)
          for h in hv]

    wd2 = wd2_ref[...]
    agg2 = jnp.dot(sum(h1), wn2_ref[...], preferred_element_type=F32) + b2_ref[...]
    h2 = [jnp.maximum(jnp.dot(h, wd2, preferred_element_type=F32) + agg2, 0.0)
          for h in h1]

    cand = h2[0]
    for h in h2[1:]:
        cand = jnp.maximum(cand, h)

    z = jnp.dot(cand, wf1_ref[...], preferred_element_type=F32) + bf1_ref[...]
    mu = jnp.mean(z, axis=-1, keepdims=True)
    var = jnp.mean((z - mu) * (z - mu), axis=-1, keepdims=True)
    z = (z - mu) * jax.lax.rsqrt(var + 1e-5) * g_ref[...] + beta_ref[...]
    z = jnp.maximum(z, 0.0)
    out_ref[...] = jnp.dot(z, wf2_ref[...], preferred_element_type=F32) + bf2_ref[...]


def _call(x2d, *ws, interpret=False):
    n = x2d.shape[0]
    full = lambda shape: pl.BlockSpec(shape, lambda i: (0, 0))
    return pl.pallas_call(
        _body,
        grid=(n // BP,),
        in_specs=[
            pl.BlockSpec((BP, V * C), lambda i: (i, 0)),
            full((C, C)), full((C, C)), full((1, C)),
            full((C, C)), full((C, C)), full((1, C)),
            full((C, C)), full((1, C)), full((1, C)), full((1, C)),
            full((C, 1)), full((1, 1)),
        ],
        out_specs=pl.BlockSpec((BP, 1), lambda i: (i, 0)),
        out_shape=jax.ShapeDtypeStruct((n, 1), F32),
        interpret=interpret,
    )(x2d, *ws)


CHUNKS = (24000, 26000)


@functools.partial(jax.jit, static_argnames=("interpret",))
def _run(x, wd1, wn1, b1, wd2, wn2, b2, wf1, bf1, g, beta, wf2, bf2,
         interpret=False):
    ws = (wd1, wn1, b1, wd2, wn2, b2, wf1, bf1, g, beta, wf2, bf2)
    outs, off = [], 0
    for n in CHUNKS:
        x2d = jax.lax.slice_in_dim(x, off, off + n, axis=0).reshape(n, V * C)
        outs.append(_call(x2d, *ws, interpret=interpret))
        off += n
    return jnp.concatenate(outs, axis=0)


def kernel(x, W1_self, W1_nbr, b1, W2_self, W2_nbr, b2, Wf1, bf1,
           ln_gamma, ln_beta, Wf2, bf2):
    return _run(
        x,
        W1_self - W1_nbr, W1_nbr, b1.reshape(1, C),
        W2_self - W2_nbr, W2_nbr, b2.reshape(1, C),
        Wf1, bf1.reshape(1, C), ln_gamma.reshape(1, C), ln_beta.reshape(1, C),
        Wf2, bf2.reshape(1, 1),
    )


# speedup vs baseline: 1.2301x; 1.2301x over previous
import functools

import jax
import jax.numpy as jnp
from jax.experimental import pallas as pl

P, V, C = 50000, 5, 128
BP = 2000
F32 = jnp.float32


def _body(x_ref, wd1_ref, wn1_ref, b1_ref, wd2_ref, wn2_ref, b2_ref,
          wf1_ref, bf1_ref, g_ref, beta_ref, wf2_ref, bf2_ref, out_ref):
    xb = x_ref[...]
    hv = [xb[:, v * C:(v + 1) * C] for v in range(V)]

    wd1 = wd1_ref[...]
    agg1 = jnp.dot(sum(hv), wn1_ref[...], preferred_element_type=F32) + b1_ref[...]
    h1 = [jnp.maximum(jnp.dot(h, wd1, preferred_element_type=F32) + agg1, 0.0)
          for h in hv]

    wd2 = wd2_ref[...]
    agg2 = jnp.dot(sum(h1), wn2_ref[...], preferred_element_type=F32) + b2_ref[...]
    h2 = [jnp.maximum(jnp.dot(h, wd2, preferred_element_type=F32) + agg2, 0.0)
          for h in h1]

    cand = h2[0]
    for h in h2[1:]:
        cand = jnp.maximum(cand, h)

    z = jnp.dot(cand, wf1_ref[...], preferred_element_type=F32) + bf1_ref[...]
    mu = jnp.mean(z, axis=-1, keepdims=True)
    var = jnp.mean((z - mu) * (z - mu), axis=-1, keepdims=True)
    z = (z - mu) * jax.lax.rsqrt(var + 1e-5) * g_ref[...] + beta_ref[...]
    z = jnp.maximum(z, 0.0)
    out_ref[...] = jnp.dot(z, wf2_ref[...], preferred_element_type=F32) + bf2_ref[...]


@functools.partial(jax.jit, static_argnames=("interpret",))
def _run(x2d, wd1, wn1, b1, wd2, wn2, b2, wf1, bf1, g, beta, wf2, bf2,
         interpret=False):
    full = lambda shape: pl.BlockSpec(shape, lambda i: (0, 0))
    return pl.pallas_call(
        _body,
        grid=(P // BP,),
        in_specs=[
            pl.BlockSpec((BP, V * C), lambda i: (i, 0)),
            full((C, C)), full((C, C)), full((1, C)),
            full((C, C)), full((C, C)), full((1, C)),
            full((C, C)), full((1, C)), full((1, C)), full((1, C)),
            full((C, 1)), full((1, 1)),
        ],
        out_specs=pl.BlockSpec((BP, 1), lambda i: (i, 0)),
        out_shape=jax.ShapeDtypeStruct((P, 1), F32),
        interpret=interpret,
    )(x2d, wd1, wn1, b1, wd2, wn2, b2, wf1, bf1, g, beta, wf2, bf2)


def kernel(x, W1_self, W1_nbr, b1, W2_self, W2_nbr, b2, Wf1, bf1,
           ln_gamma, ln_beta, Wf2, bf2):
    return _run(
        x.reshape(P, V * C),
        W1_self - W1_nbr, W1_nbr, b1.reshape(1, C),
        W2_self - W2_nbr, W2_nbr, b2.reshape(1, C),
        Wf1, bf1.reshape(1, C), ln_gamma.reshape(1, C), ln_beta.reshape(1, C),
        Wf2, bf2.reshape(1, 1),
    )
